# baseline (speedup 1.0000x reference)
import functools

import jax
import jax.numpy as jnp
from jax import lax
from jax.experimental import pallas as pl
from jax.experimental.pallas import tpu as pltpu
from jax.experimental.pallas import tpu_sc as plsc


def kernel(x, perm):
    B, S, F = x.shape
    info = plsc.get_sparse_core_info()
    NC, NS, L = info.num_cores, info.num_subcores, info.num_lanes
    NW = NC * NS
    R = B * S
    rows_per_w = R // NW
    w_per_b = S // rows_per_w
    BLK = 4
    NBUF = 6
    n_blocks = rows_per_w // BLK
    n_iters = -(-n_blocks // NBUF)

    mesh = plsc.VectorSubcoreMesh(core_axis_name="c", subcore_axis_name="s")

    @functools.partial(
        pl.kernel,
        mesh=mesh,
        out_type=jax.ShapeDtypeStruct((B, S, F), jnp.float32),
        scratch_types=[
            pltpu.VMEM((F,), jnp.int32),
            *(pltpu.VMEM((BLK, F), jnp.float32),) * NBUF,
            *(pltpu.VMEM((BLK, F), jnp.float32),) * NBUF,
            *(pltpu.SemaphoreType.DMA,) * (2 * NBUF),
        ],
        compiler_params=pltpu.CompilerParams(needs_layout_passes=False),
    )
    def run(x_hbm, perm_hbm, out_hbm, idx_v, *bufs):
        ins = bufs[:NBUF]
        outs = bufs[NBUF:2 * NBUF]
        isems = bufs[2 * NBUF:3 * NBUF]
        osems = bufs[3 * NBUF:]
        wid = lax.axis_index("s") * NC + lax.axis_index("c")
        bidx = wid // w_per_b
        row0 = (wid % w_per_b) * rows_per_w

        def in_copy(b, ph):
            return pltpu.make_async_copy(
                x_hbm.at[bidx, pl.ds(row0 + b * BLK, BLK)],
                ins[ph], isems[ph])

        def out_copy(b, ph):
            return pltpu.make_async_copy(
                outs[ph], out_hbm.at[bidx, pl.ds(row0 + b * BLK, BLK)],
                osems[ph])

        pltpu.sync_copy(perm_hbm, idx_v)
        for ph in range(NBUF):
            in_copy(ph, ph).start()
        rows = [jnp.full((L,), r, jnp.int32) for r in range(BLK)]

        def grp_body(bb, carry):
            for ph in range(NBUF):
                b = bb * NBUF + ph

                @pl.when(jnp.logical_and(b >= NBUF, b < n_blocks))
                def _():
                    out_copy(b - NBUF, ph).wait()

                @pl.when(b < n_blocks)
                def _():
                    in_copy(b, ph).wait()
                    in_v, out_v = ins[ph], outs[ph]

                    @plsc.parallel_loop(0, F // L, unroll=4)
                    def _(g):
                        idx = idx_v[pl.ds(g * L, L)]
                        for r in range(BLK):
                            vals = plsc.load_gather(in_v, [rows[r], idx])
                            out_v[r, pl.ds(g * L, L)] = vals

                    out_copy(b, ph).start()

                @pl.when(b + NBUF < n_blocks)
                def _():
                    in_copy(b + NBUF, ph).start()
            return carry

        lax.fori_loop(0, n_iters, grp_body, 0)
        for b in range(n_blocks - NBUF, n_blocks):
            out_copy(b, b % NBUF).wait()

    return run(x, perm)


# speedup vs baseline: 1.0022x; 1.0022x over previous
import functools

import jax
import jax.numpy as jnp
from jax import lax
from jax.experimental import pallas as pl
from jax.experimental.pallas import tpu as pltpu
from jax.experimental.pallas import tpu_sc as plsc


def kernel(x, perm):
    B, S, F = x.shape
    info = plsc.get_sparse_core_info()
    NC, NS, L = info.num_cores, info.num_subcores, info.num_lanes
    NW = NC * NS
    R = B * S
    rows_per_w = R // NW
    w_per_b = S // rows_per_w
    BLK = 8
    NBUF = 3
    n_blocks = rows_per_w // BLK
    n_iters = -(-n_blocks // NBUF)

    mesh = plsc.VectorSubcoreMesh(core_axis_name="c", subcore_axis_name="s")

    @functools.partial(
        pl.kernel,
        mesh=mesh,
        out_type=jax.ShapeDtypeStruct((B, S, F), jnp.float32),
        scratch_types=[
            pltpu.VMEM((F,), jnp.int32),
            *(pltpu.VMEM((BLK, F), jnp.float32),) * NBUF,
            *(pltpu.VMEM((BLK, F), jnp.float32),) * NBUF,
            *(pltpu.SemaphoreType.DMA,) * (2 * NBUF),
        ],
        compiler_params=pltpu.CompilerParams(needs_layout_passes=False),
    )
    def run(x_hbm, perm_hbm, out_hbm, idx_v, *bufs):
        ins = bufs[:NBUF]
        outs = bufs[NBUF:2 * NBUF]
        isems = bufs[2 * NBUF:3 * NBUF]
        osems = bufs[3 * NBUF:]
        wid = lax.axis_index("s") * NC + lax.axis_index("c")
        bidx = wid // w_per_b
        row0 = (wid % w_per_b) * rows_per_w

        def in_copy(b, ph):
            return pltpu.make_async_copy(
                x_hbm.at[bidx, pl.ds(row0 + b * BLK, BLK)],
                ins[ph], isems[ph])

        def out_copy(b, ph):
            return pltpu.make_async_copy(
                outs[ph], out_hbm.at[bidx, pl.ds(row0 + b * BLK, BLK)],
                osems[ph])

        pltpu.sync_copy(perm_hbm, idx_v)
        for ph in range(NBUF):
            in_copy(ph, ph).start()
        rows = [jnp.full((L,), r, jnp.int32) for r in range(BLK)]

        def grp_body(bb, carry):
            for ph in range(NBUF):
                b = bb * NBUF + ph

                @pl.when(jnp.logical_and(b >= NBUF, b < n_blocks))
                def _():
                    out_copy(b - NBUF, ph).wait()

                @pl.when(b < n_blocks)
                def _():
                    in_copy(b, ph).wait()
                    in_v, out_v = ins[ph], outs[ph]

                    @plsc.parallel_loop(0, F // L, unroll=4)
                    def _(g):
                        idx = idx_v[pl.ds(g * L, L)]
                        for r in range(BLK):
                            vals = plsc.load_gather(in_v, [rows[r], idx])
                            out_v[r, pl.ds(g * L, L)] = vals

                    out_copy(b, ph).start()

                @pl.when(b + NBUF < n_blocks)
                def _():
                    in_copy(b + NBUF, ph).start()
            return carry

        lax.fori_loop(0, n_iters, grp_body, 0)
        for b in range(n_blocks - NBUF, n_blocks):
            out_copy(b, b % NBUF).wait()

    return run(x, perm)
